# baseline (speedup 1.0000x reference)
import functools

import jax
import jax.numpy as jnp
from jax import lax
from jax.experimental import pallas as pl
from jax.experimental.pallas import tpu as pltpu
from jax.experimental.pallas import tpu_sc as plsc

HIDDEN = 2048
NUM_WORKERS = 32
CHUNK = 8
NBUF = 4


def _emb_body(idx_hbm, table_hbm, out_hbm, idx_v, *rest, bpw, n_chunks):
    bufs = rest[:NBUF]
    sems = rest[NBUF:2 * NBUF]
    wid = lax.axis_index("s") * 2 + lax.axis_index("c")
    base = wid * bpw
    pltpu.sync_copy(idx_hbm.at[pl.ds(base, bpw)], idx_v)

    def gather(g, b):
        return pltpu.make_async_copy(
            table_hbm.at[idx_v.at[pl.ds(g * CHUNK, CHUNK)]], bufs[b], sems[b]
        )

    for b in range(NBUF):
        gather(b, b).start()

    def body(k, carry):
        for b in range(NBUF):
            g = NBUF * k + b
            gather(g, b).wait()
            pltpu.sync_copy(bufs[b], out_hbm.at[pl.ds(base + g * CHUNK, CHUNK)])

            @pl.when(g + NBUF < n_chunks)
            def _():
                gather(g + NBUF, b).start()

        return carry

    lax.fori_loop(0, n_chunks // NBUF, body, 0)


def kernel(input_ids, table):
    b, t = input_ids.shape
    n = b * t
    idx = input_ids.reshape(n).astype(jnp.int32)
    bpw = n // NUM_WORKERS
    n_chunks = bpw // CHUNK

    mesh = plsc.VectorSubcoreMesh(core_axis_name="c", subcore_axis_name="s")
    emb = pl.kernel(
        functools.partial(_emb_body, bpw=bpw, n_chunks=n_chunks),
        mesh=mesh,
        out_type=jax.ShapeDtypeStruct((n, HIDDEN), jnp.float32),
        scratch_types=(
            [pltpu.VMEM((bpw,), jnp.int32)]
            + [pltpu.VMEM((CHUNK, HIDDEN), jnp.float32)] * NBUF
            + [pltpu.SemaphoreType.DMA] * NBUF
        ),
    )
    out = emb(idx, table)
    return out.reshape(b, t, HIDDEN)
